# baseline (speedup 1.0000x reference)
import jax
import jax.numpy as jnp
from jax.experimental import pallas as pl
from jax.experimental.pallas import tpu as pltpu

N = 32768
B = 16
TILE = 4096
GRID = N // TILE


def _body(pos_ref, batch_ref, h0_ref,
          FW1_ref, Fb1_ref, FW2_ref, Fb2_ref,
          GW1_ref, Gb1_ref, GW2_ref, Gb2_ref,
          AWobs_ref, AWh_ref, Abh_ref, AWl_ref, Abl_ref, AWa_ref, Aba_ref,
          aff_ref, recon_ref, coh_ref, spatial_ref, action_ref, hnext_ref,
          acc_seg, acc_cnt):
    i = pl.program_id(0)
    bf = jnp.bfloat16

    pos = pos_ref[...]
    h1 = jnp.maximum(
        jnp.dot(pos, FW1_ref[...],
                preferred_element_type=jnp.float32).astype(bf)
        + Fb1_ref[...].astype(bf), 0)
    aff = jnp.dot(h1, FW2_ref[...].astype(bf),
                  preferred_element_type=jnp.float32) + Fb2_ref[...]
    affb = aff.astype(bf)
    g1 = jnp.maximum(
        jnp.dot(affb, GW1_ref[...].astype(bf),
                preferred_element_type=jnp.float32).astype(bf)
        + Gb1_ref[...].astype(bf), 0)
    recon = jnp.dot(g1, GW2_ref[...].astype(bf),
                    preferred_element_type=jnp.float32) + Gb2_ref[...]
    d = pos - recon
    d_t = jax.lax.transpose(d, (1, 0))
    err_row = jnp.sum(d_t * d_t, axis=0, keepdims=True)

    aff_ref[...] = aff
    recon_ref[...] = recon
    spatial_ref[0] = err_row

    one_hot_t = (batch_ref[0] == jax.lax.broadcasted_iota(
        jnp.int32, (B, TILE), 0)).astype(bf)
    seg_aff = jnp.dot(one_hot_t, affb,
                      preferred_element_type=jnp.float32)
    one_hot_f = one_hot_t.astype(jnp.float32)
    seg_err = jnp.sum(one_hot_f * err_row, axis=1, keepdims=True)
    cnt = jnp.sum(one_hot_f, axis=1, keepdims=True)
    seg = jnp.concatenate([seg_aff, cnt], axis=1)

    @pl.when(i == 0)
    def _init():
        acc_seg[...] = seg
        acc_cnt[...] = seg_err

    @pl.when(i > 0)
    def _accum():
        acc_seg[...] += seg
        acc_cnt[...] += seg_err

    @pl.when(i == GRID - 1)
    def _final():
        acc = acc_seg[...]
        counts = acc[:, 16:17]
        safe = jnp.maximum(counts, 1.0)
        nonzero = counts > 0.0
        coh_ref[...] = jnp.where(nonzero, acc_cnt[...] / safe, 0.0)
        batch_aff = jnp.where(nonzero, acc[:, :16] / safe, 0.0)
        h_next = jnp.tanh(
            jnp.dot(batch_aff, AWobs_ref[...], preferred_element_type=jnp.float32)
            + jnp.dot(h0_ref[...], AWh_ref[...], preferred_element_type=jnp.float32)
            + Abh_ref[...])
        latent = jnp.maximum(
            jnp.dot(h_next, AWl_ref[...], preferred_element_type=jnp.float32)
            + Abl_ref[...], 0.0)
        action_ref[...] = jnp.dot(
            latent, AWa_ref[...], preferred_element_type=jnp.float32) + Aba_ref[...]
        hnext_ref[...] = h_next


def kernel(pos, batch, agent_state_h, coherence_signal_prev, coherence_spatial_prev,
           F_W1, F_b1, F_W2, F_b2, G_W1, G_b1, G_W2, G_b2,
           A_Wobs, A_Wh, A_bh, A_Wl, A_bl, A_Wa, A_ba):
    del coherence_signal_prev, coherence_spatial_prev

    batch3 = batch.reshape(GRID, 1, TILE)
    row = lambda v: v.reshape(1, -1)
    tile_spec = lambda w: pl.BlockSpec((TILE, w), lambda i: (i, 0))
    full = lambda a: pl.BlockSpec(a.shape, lambda i: (0,) * a.ndim)

    out_shapes = (
        jax.ShapeDtypeStruct((N, 16), jnp.float32),
        jax.ShapeDtypeStruct((N, 3), jnp.float32),
        jax.ShapeDtypeStruct((B, 1), jnp.float32),
        jax.ShapeDtypeStruct((GRID, 1, TILE), jnp.float32),
        jax.ShapeDtypeStruct((B, 8), jnp.float32),
        jax.ShapeDtypeStruct((B, 64), jnp.float32),
    )

    small = (agent_state_h, F_W1, row(F_b1), F_W2, row(F_b2),
             G_W1, row(G_b1), G_W2, row(G_b2),
             A_Wobs, A_Wh, row(A_bh), A_Wl, row(A_bl), A_Wa, row(A_ba))

    outs = pl.pallas_call(
        _body,
        grid=(GRID,),
        in_specs=[tile_spec(3), pl.BlockSpec((1, 1, TILE), lambda i: (i, 0, 0))]
                 + [full(a) for a in small],
        out_specs=[tile_spec(16), tile_spec(3),
                   pl.BlockSpec((B, 1), lambda i: (0, 0)),
                   pl.BlockSpec((1, 1, TILE), lambda i: (i, 0, 0)),
                   pl.BlockSpec((B, 8), lambda i: (0, 0)),
                   pl.BlockSpec((B, 64), lambda i: (0, 0))],
        out_shape=out_shapes,
        scratch_shapes=[pltpu.VMEM((B, 17), jnp.float32),
                        pltpu.VMEM((B, 1), jnp.float32)],
        compiler_params=pltpu.CompilerParams(
            dimension_semantics=("arbitrary",)),
    )(pos, batch3, *small)

    affordances, recon, coh, spatial, action, h_next = outs
    return (affordances, recon, coh, spatial.reshape(N), action, h_next)


# speedup vs baseline: 1.0221x; 1.0221x over previous
import jax
import jax.numpy as jnp
from jax.experimental import pallas as pl
from jax.experimental.pallas import tpu as pltpu

N = 32768
B = 16
TILE = 16384
GRID = N // TILE


def _body(pos_ref, batch_ref, h0_ref,
          FW1_ref, Fb1_ref, FW2_ref, Fb2_ref,
          GW1_ref, Gb1_ref, GW2_ref, Gb2_ref,
          AWobs_ref, AWh_ref, Abh_ref, AWl_ref, Abl_ref, AWa_ref, Aba_ref,
          aff_ref, recon_ref, coh_ref, spatial_ref, action_ref, hnext_ref,
          acc_seg, acc_cnt):
    i = pl.program_id(0)
    bf = jnp.bfloat16

    pos = pos_ref[...]
    h1 = jnp.maximum(
        jnp.dot(pos, FW1_ref[...],
                preferred_element_type=jnp.float32).astype(bf)
        + Fb1_ref[...].astype(bf), 0)
    aff = jnp.dot(h1, FW2_ref[...].astype(bf),
                  preferred_element_type=jnp.float32) + Fb2_ref[...]
    affb = aff.astype(bf)
    g1 = jnp.maximum(
        jnp.dot(affb, GW1_ref[...].astype(bf),
                preferred_element_type=jnp.float32).astype(bf)
        + Gb1_ref[...].astype(bf), 0)
    recon = jnp.dot(g1, GW2_ref[...].astype(bf),
                    preferred_element_type=jnp.float32) + Gb2_ref[...]
    d = pos - recon
    d_t = jax.lax.transpose(d, (1, 0))
    err_row = jnp.sum(d_t * d_t, axis=0, keepdims=True)

    aff_ref[...] = aff
    recon_ref[...] = recon
    spatial_ref[0] = err_row

    one_hot_t = (batch_ref[0] == jax.lax.broadcasted_iota(
        jnp.int32, (B, TILE), 0)).astype(bf)
    seg_aff = jnp.dot(one_hot_t, affb,
                      preferred_element_type=jnp.float32)
    one_hot_f = one_hot_t.astype(jnp.float32)
    seg_err = jnp.sum(one_hot_f * err_row, axis=1, keepdims=True)
    cnt = jnp.sum(one_hot_f, axis=1, keepdims=True)
    seg = jnp.concatenate([seg_aff, cnt], axis=1)

    @pl.when(i == 0)
    def _init():
        acc_seg[...] = seg
        acc_cnt[...] = seg_err

    @pl.when(i > 0)
    def _accum():
        acc_seg[...] += seg
        acc_cnt[...] += seg_err

    @pl.when(i == GRID - 1)
    def _final():
        acc = acc_seg[...]
        counts = acc[:, 16:17]
        safe = jnp.maximum(counts, 1.0)
        nonzero = counts > 0.0
        coh_ref[...] = jnp.where(nonzero, acc_cnt[...] / safe, 0.0)
        batch_aff = jnp.where(nonzero, acc[:, :16] / safe, 0.0)
        h_next = jnp.tanh(
            jnp.dot(batch_aff, AWobs_ref[...], preferred_element_type=jnp.float32)
            + jnp.dot(h0_ref[...], AWh_ref[...], preferred_element_type=jnp.float32)
            + Abh_ref[...])
        latent = jnp.maximum(
            jnp.dot(h_next, AWl_ref[...], preferred_element_type=jnp.float32)
            + Abl_ref[...], 0.0)
        action_ref[...] = jnp.dot(
            latent, AWa_ref[...], preferred_element_type=jnp.float32) + Aba_ref[...]
        hnext_ref[...] = h_next


def kernel(pos, batch, agent_state_h, coherence_signal_prev, coherence_spatial_prev,
           F_W1, F_b1, F_W2, F_b2, G_W1, G_b1, G_W2, G_b2,
           A_Wobs, A_Wh, A_bh, A_Wl, A_bl, A_Wa, A_ba):
    del coherence_signal_prev, coherence_spatial_prev

    batch3 = batch.reshape(GRID, 1, TILE)
    row = lambda v: v.reshape(1, -1)
    tile_spec = lambda w: pl.BlockSpec((TILE, w), lambda i: (i, 0))
    full = lambda a: pl.BlockSpec(a.shape, lambda i: (0,) * a.ndim)

    out_shapes = (
        jax.ShapeDtypeStruct((N, 16), jnp.float32),
        jax.ShapeDtypeStruct((N, 3), jnp.float32),
        jax.ShapeDtypeStruct((B, 1), jnp.float32),
        jax.ShapeDtypeStruct((GRID, 1, TILE), jnp.float32),
        jax.ShapeDtypeStruct((B, 8), jnp.float32),
        jax.ShapeDtypeStruct((B, 64), jnp.float32),
    )

    small = (agent_state_h, F_W1, row(F_b1), F_W2, row(F_b2),
             G_W1, row(G_b1), G_W2, row(G_b2),
             A_Wobs, A_Wh, row(A_bh), A_Wl, row(A_bl), A_Wa, row(A_ba))

    outs = pl.pallas_call(
        _body,
        grid=(GRID,),
        in_specs=[tile_spec(3), pl.BlockSpec((1, 1, TILE), lambda i: (i, 0, 0))]
                 + [full(a) for a in small],
        out_specs=[tile_spec(16), tile_spec(3),
                   pl.BlockSpec((B, 1), lambda i: (0, 0)),
                   pl.BlockSpec((1, 1, TILE), lambda i: (i, 0, 0)),
                   pl.BlockSpec((B, 8), lambda i: (0, 0)),
                   pl.BlockSpec((B, 64), lambda i: (0, 0))],
        out_shape=out_shapes,
        scratch_shapes=[pltpu.VMEM((B, 17), jnp.float32),
                        pltpu.VMEM((B, 1), jnp.float32)],
        compiler_params=pltpu.CompilerParams(
            dimension_semantics=("arbitrary",)),
    )(pos, batch3, *small)

    affordances, recon, coh, spatial, action, h_next = outs
    return (affordances, recon, coh, spatial.reshape(N), action, h_next)


# speedup vs baseline: 1.0264x; 1.0042x over previous
import jax
import jax.numpy as jnp
from jax.experimental import pallas as pl
from jax.experimental.pallas import tpu as pltpu

N = 32768
B = 16
TILE = 8192
GRID = N // TILE


def _body(pos_ref, batch_ref, h0_ref,
          FW1_ref, Fb1_ref, FW2_ref, Fb2_ref,
          GW1_ref, Gb1_ref, GW2_ref, Gb2_ref,
          AWobs_ref, AWh_ref, Abh_ref, AWl_ref, Abl_ref, AWa_ref, Aba_ref,
          aff_ref, recon_ref, coh_ref, spatial_ref, action_ref, hnext_ref,
          acc_seg, acc_cnt):
    i = pl.program_id(0)
    bf = jnp.bfloat16

    pos = pos_ref[...]
    h1 = jnp.maximum(
        jnp.dot(pos, FW1_ref[...],
                preferred_element_type=jnp.float32).astype(bf)
        + Fb1_ref[...].astype(bf), 0)
    aff = jnp.dot(h1, FW2_ref[...].astype(bf),
                  preferred_element_type=jnp.float32) + Fb2_ref[...]
    affb = aff.astype(bf)
    g1 = jnp.maximum(
        jnp.dot(affb, GW1_ref[...].astype(bf),
                preferred_element_type=jnp.float32).astype(bf)
        + Gb1_ref[...].astype(bf), 0)
    recon = jnp.dot(g1, GW2_ref[...].astype(bf),
                    preferred_element_type=jnp.float32) + Gb2_ref[...]
    d = pos - recon
    d_t = jax.lax.transpose(d, (1, 0))
    err_row = jnp.sum(d_t * d_t, axis=0, keepdims=True)

    aff_ref[...] = aff
    recon_ref[...] = recon
    spatial_ref[...] = err_row.reshape(TILE)

    one_hot_t = (batch_ref[...].reshape(1, TILE) == jax.lax.broadcasted_iota(
        jnp.int32, (B, TILE), 0)).astype(bf)
    seg_aff = jnp.dot(one_hot_t, affb,
                      preferred_element_type=jnp.float32)
    one_hot_f = one_hot_t.astype(jnp.float32)
    seg_err = jnp.sum(one_hot_f * err_row, axis=1, keepdims=True)
    cnt = jnp.sum(one_hot_f, axis=1, keepdims=True)
    seg = jnp.concatenate([seg_aff, cnt], axis=1)

    @pl.when(i == 0)
    def _init():
        acc_seg[...] = seg
        acc_cnt[...] = seg_err

    @pl.when(i > 0)
    def _accum():
        acc_seg[...] += seg
        acc_cnt[...] += seg_err

    @pl.when(i == GRID - 1)
    def _final():
        acc = acc_seg[...]
        counts = acc[:, 16:17]
        safe = jnp.maximum(counts, 1.0)
        nonzero = counts > 0.0
        coh_ref[...] = jnp.where(nonzero, acc_cnt[...] / safe, 0.0)
        batch_aff = jnp.where(nonzero, acc[:, :16] / safe, 0.0)
        h_next = jnp.tanh(
            jnp.dot(batch_aff, AWobs_ref[...], preferred_element_type=jnp.float32)
            + jnp.dot(h0_ref[...], AWh_ref[...], preferred_element_type=jnp.float32)
            + Abh_ref[...])
        latent = jnp.maximum(
            jnp.dot(h_next, AWl_ref[...], preferred_element_type=jnp.float32)
            + Abl_ref[...], 0.0)
        action_ref[...] = jnp.dot(
            latent, AWa_ref[...], preferred_element_type=jnp.float32) + Aba_ref[...]
        hnext_ref[...] = h_next


def kernel(pos, batch, agent_state_h, coherence_signal_prev, coherence_spatial_prev,
           F_W1, F_b1, F_W2, F_b2, G_W1, G_b1, G_W2, G_b2,
           A_Wobs, A_Wh, A_bh, A_Wl, A_bl, A_Wa, A_ba):
    del coherence_signal_prev, coherence_spatial_prev

    row = lambda v: v.reshape(1, -1)
    tile_spec = lambda w: pl.BlockSpec((TILE, w), lambda i: (i, 0))
    full = lambda a: pl.BlockSpec(a.shape, lambda i: (0,) * a.ndim)

    out_shapes = (
        jax.ShapeDtypeStruct((N, 16), jnp.float32),
        jax.ShapeDtypeStruct((N, 3), jnp.float32),
        jax.ShapeDtypeStruct((B, 1), jnp.float32),
        jax.ShapeDtypeStruct((N,), jnp.float32),
        jax.ShapeDtypeStruct((B, 8), jnp.float32),
        jax.ShapeDtypeStruct((B, 64), jnp.float32),
    )

    small = (agent_state_h, F_W1, row(F_b1), F_W2, row(F_b2),
             G_W1, row(G_b1), G_W2, row(G_b2),
             A_Wobs, A_Wh, row(A_bh), A_Wl, row(A_bl), A_Wa, row(A_ba))

    outs = pl.pallas_call(
        _body,
        grid=(GRID,),
        in_specs=[tile_spec(3), pl.BlockSpec((TILE,), lambda i: (i,))]
                 + [full(a) for a in small],
        out_specs=[tile_spec(16), tile_spec(3),
                   pl.BlockSpec((B, 1), lambda i: (0, 0)),
                   pl.BlockSpec((TILE,), lambda i: (i,)),
                   pl.BlockSpec((B, 8), lambda i: (0, 0)),
                   pl.BlockSpec((B, 64), lambda i: (0, 0))],
        out_shape=out_shapes,
        scratch_shapes=[pltpu.VMEM((B, 17), jnp.float32),
                        pltpu.VMEM((B, 1), jnp.float32)],
        compiler_params=pltpu.CompilerParams(
            dimension_semantics=("arbitrary",)),
    )(pos, batch, *small)

    affordances, recon, coh, spatial, action, h_next = outs
    return (affordances, recon, coh, spatial, action, h_next)
